# baseline (speedup 1.0000x reference)
import functools

import jax
import jax.numpy as jnp
from jax import lax
from jax.experimental import pallas as pl
from jax.experimental.pallas import tpu as pltpu
from jax.experimental.pallas import tpu_sc as plsc

DIM = 128
ROWS = 8192
N_IDX = 4096 * 50
GROUP = 80
_info = plsc.get_sparse_core_info()
NC = _info.num_cores
NS = _info.num_subcores
NW = NC * NS
PER_W = N_IDX // NW
NGROUP = PER_W // GROUP
NBUF = 5
LA = 3
NB = NGROUP // NBUF
FB = 2
STAGE = ROWS // NS


@functools.partial(
    pl.kernel,
    out_type=jax.ShapeDtypeStruct((N_IDX, DIM), jnp.float32),
    mesh=plsc.VectorSubcoreMesh(core_axis_name="c", subcore_axis_name="s"),
    scratch_types=[
        pltpu.VMEM_SHARED((ROWS, DIM), jnp.float32),
        pltpu.VMEM((NGROUP, GROUP), jnp.int32),
        [pltpu.VMEM((GROUP, DIM), jnp.float32)] * NBUF,
        [pltpu.SemaphoreType.DMA] * NBUF,
        [pltpu.SemaphoreType.DMA] * NBUF,
        pltpu.SemaphoreType.DMA,
    ],
)
def _gather_kernel(table_hbm, idx_hbm, out_hbm, table_sp, idx_v, rows, gsem,
                   ssem, tsem):
    cid = lax.axis_index("c")
    sid = lax.axis_index("s")
    wid = sid * NC + cid
    row_base = wid * NGROUP

    stage_desc = pltpu.async_copy(
        table_hbm.at[pl.ds(sid * STAGE, STAGE)],
        table_sp.at[pl.ds(sid * STAGE, STAGE)],
        tsem,
    )
    pltpu.sync_copy(idx_hbm.at[wid], idx_v)

    def start_gather_hbm(j, b):
        pltpu.async_copy(table_hbm.at[idx_v.at[j]], rows[b], gsem[b])

    def start_gather(j, b):
        pltpu.async_copy(table_sp.at[idx_v.at[j]], rows[b], gsem[b])

    def wait_gather(b):
        pltpu.make_async_copy(
            table_sp.at[pl.ds(0, GROUP)], rows[b], gsem[b]
        ).wait()

    def start_store(j, b):
        pltpu.async_copy(
            rows[b], out_hbm.at[pl.ds((row_base + j) * GROUP, GROUP)], ssem[b]
        )

    def wait_store(b):
        pltpu.make_async_copy(
            rows[b], out_hbm.at[pl.ds(0, GROUP)], ssem[b]
        ).wait()

    for j in range(LA):
        start_gather_hbm(j, j)

    for b in range(NBUF):
        jn = b + LA
        if jn >= NBUF:
            wait_store(jn % NBUF)
        start_gather_hbm(jn, jn % NBUF)
        wait_gather(b)
        start_store(b, b)

    @pl.loop(1, FB)
    def _(g):
        j0 = g * NBUF
        for b in range(NBUF):
            bn = (b + LA) % NBUF
            wait_store(bn)
            start_gather_hbm(j0 + b + LA, bn)
            wait_gather(b)
            start_store(j0 + b, b)

    stage_desc.wait()
    plsc.subcore_barrier()

    @pl.loop(FB, NB - 1)
    def _(g):
        j0 = g * NBUF
        for b in range(NBUF):
            bn = (b + LA) % NBUF
            wait_store(bn)
            start_gather(j0 + b + LA, bn)
            wait_gather(b)
            start_store(j0 + b, b)

    j0 = (NB - 1) * NBUF
    for b in range(NBUF):
        if b < NBUF - LA:
            bn = (b + LA) % NBUF
            wait_store(bn)
            start_gather(j0 + b + LA, bn)
        wait_gather(b)
        start_store(j0 + b, b)

    for b in range(NBUF):
        wait_store(b)


def kernel(positions, encodings):
    idx = positions.reshape(NW, NGROUP, GROUP).astype(jnp.int32)
    out = _gather_kernel(encodings, idx)
    return out.reshape(positions.shape[0], 1, positions.shape[1], DIM)


# speedup vs baseline: 1.0427x; 1.0427x over previous
import functools

import jax
import jax.numpy as jnp
from jax import lax
from jax.experimental import pallas as pl
from jax.experimental.pallas import tpu as pltpu
from jax.experimental.pallas import tpu_sc as plsc

DIM = 128
ROWS = 8192
N_IDX = 4096 * 50
GROUP = 80
_info = plsc.get_sparse_core_info()
NC = _info.num_cores
NS = _info.num_subcores
NW = NC * NS
PER_W = N_IDX // NW
NGROUP = PER_W // GROUP
NBUF = 5
LA = 3
NB = NGROUP // NBUF
FB = 1
STAGE = ROWS // NS


@functools.partial(
    pl.kernel,
    out_type=jax.ShapeDtypeStruct((N_IDX, DIM), jnp.float32),
    mesh=plsc.VectorSubcoreMesh(core_axis_name="c", subcore_axis_name="s"),
    scratch_types=[
        pltpu.VMEM_SHARED((ROWS, DIM), jnp.float32),
        pltpu.VMEM((NGROUP, GROUP), jnp.int32),
        [pltpu.VMEM((GROUP, DIM), jnp.float32)] * NBUF,
        [pltpu.SemaphoreType.DMA] * NBUF,
        [pltpu.SemaphoreType.DMA] * NBUF,
        pltpu.SemaphoreType.DMA,
    ],
)
def _gather_kernel(table_hbm, idx_hbm, out_hbm, table_sp, idx_v, rows, gsem,
                   ssem, tsem):
    cid = lax.axis_index("c")
    sid = lax.axis_index("s")
    wid = sid * NC + cid
    row_base = wid * NGROUP

    stage_desc = pltpu.async_copy(
        table_hbm.at[pl.ds(sid * STAGE, STAGE)],
        table_sp.at[pl.ds(sid * STAGE, STAGE)],
        tsem,
    )
    pltpu.sync_copy(idx_hbm.at[wid], idx_v)

    def start_gather_hbm(j, b):
        pltpu.async_copy(table_hbm.at[idx_v.at[j]], rows[b], gsem[b])

    def start_gather(j, b):
        pltpu.async_copy(table_sp.at[idx_v.at[j]], rows[b], gsem[b])

    def wait_gather(b):
        pltpu.make_async_copy(
            table_sp.at[pl.ds(0, GROUP)], rows[b], gsem[b]
        ).wait()

    def start_store(j, b):
        pltpu.async_copy(
            rows[b], out_hbm.at[pl.ds((row_base + j) * GROUP, GROUP)], ssem[b]
        )

    def wait_store(b):
        pltpu.make_async_copy(
            rows[b], out_hbm.at[pl.ds(0, GROUP)], ssem[b]
        ).wait()

    for j in range(LA):
        start_gather_hbm(j, j)

    for b in range(NBUF):
        jn = b + LA
        if jn >= NBUF:
            wait_store(jn % NBUF)
        start_gather_hbm(jn, jn % NBUF)
        wait_gather(b)
        start_store(b, b)

    @pl.loop(1, FB)
    def _(g):
        j0 = g * NBUF
        for b in range(NBUF):
            bn = (b + LA) % NBUF
            wait_store(bn)
            start_gather_hbm(j0 + b + LA, bn)
            wait_gather(b)
            start_store(j0 + b, b)

    stage_desc.wait()
    plsc.subcore_barrier()

    @pl.loop(FB, NB - 1)
    def _(g):
        j0 = g * NBUF
        for b in range(NBUF):
            bn = (b + LA) % NBUF
            wait_store(bn)
            start_gather(j0 + b + LA, bn)
            wait_gather(b)
            start_store(j0 + b, b)

    j0 = (NB - 1) * NBUF
    for b in range(NBUF):
        if b < NBUF - LA:
            bn = (b + LA) % NBUF
            wait_store(bn)
            start_gather(j0 + b + LA, bn)
        wait_gather(b)
        start_store(j0 + b, b)

    for b in range(NBUF):
        wait_store(b)


def kernel(positions, encodings):
    idx = positions.reshape(NW, NGROUP, GROUP).astype(jnp.int32)
    out = _gather_kernel(encodings, idx)
    return out.reshape(positions.shape[0], 1, positions.shape[1], DIM)


# speedup vs baseline: 1.0642x; 1.0206x over previous
import functools

import jax
import jax.numpy as jnp
from jax import lax
from jax.experimental import pallas as pl
from jax.experimental.pallas import tpu as pltpu
from jax.experimental.pallas import tpu_sc as plsc

DIM = 128
ROWS = 8192
N_IDX = 4096 * 50
GROUP = 80
_info = plsc.get_sparse_core_info()
NC = _info.num_cores
NS = _info.num_subcores
NW = NC * NS
PER_W = N_IDX // NW
NGROUP = PER_W // GROUP
NBUF = 5
LA = 3
NB = NGROUP // NBUF
STAGE = ROWS // NS


@functools.partial(
    pl.kernel,
    out_type=jax.ShapeDtypeStruct((N_IDX, DIM), jnp.float32),
    mesh=plsc.VectorSubcoreMesh(core_axis_name="c", subcore_axis_name="s"),
    scratch_types=[
        pltpu.VMEM_SHARED((ROWS, DIM), jnp.float32),
        pltpu.VMEM((NGROUP, GROUP), jnp.int32),
        [pltpu.VMEM((GROUP, DIM), jnp.float32)] * NBUF,
        [pltpu.SemaphoreType.DMA] * NBUF,
        [pltpu.SemaphoreType.DMA] * NBUF,
        pltpu.SemaphoreType.DMA,
    ],
)
def _gather_kernel(table_hbm, idx_hbm, out_hbm, table_sp, idx_v, rows, gsem,
                   ssem, tsem):
    cid = lax.axis_index("c")
    sid = lax.axis_index("s")
    wid = sid * NC + cid
    row_base = wid * NGROUP

    stage_desc = pltpu.async_copy(
        table_hbm.at[pl.ds(sid * STAGE, STAGE)],
        table_sp.at[pl.ds(sid * STAGE, STAGE)],
        tsem,
    )
    pltpu.sync_copy(idx_hbm.at[wid], idx_v)
    stage_desc.wait()
    plsc.subcore_barrier()

    def start_gather(j, b):
        pltpu.async_copy(table_sp.at[idx_v.at[j]], rows[b], gsem[b])

    def wait_gather(b):
        pltpu.make_async_copy(
            table_sp.at[pl.ds(0, GROUP)], rows[b], gsem[b]
        ).wait()

    def start_store(j, b):
        pltpu.async_copy(
            rows[b], out_hbm.at[pl.ds((row_base + j) * GROUP, GROUP)], ssem[b]
        )

    def wait_store(b):
        pltpu.make_async_copy(
            rows[b], out_hbm.at[pl.ds(0, GROUP)], ssem[b]
        ).wait()

    for j in range(LA):
        start_gather(j, j)

    for b in range(NBUF):
        jn = b + LA
        if jn >= NBUF:
            wait_store(jn % NBUF)
        start_gather(jn, jn % NBUF)
        wait_gather(b)
        start_store(b, b)

    @pl.loop(1, NB - 1)
    def _(g):
        j0 = g * NBUF
        for b in range(NBUF):
            bn = (b + LA) % NBUF
            wait_store(bn)
            start_gather(j0 + b + LA, bn)
            wait_gather(b)
            start_store(j0 + b, b)

    j0 = (NB - 1) * NBUF
    for b in range(NBUF):
        if b < NBUF - LA:
            bn = (b + LA) % NBUF
            wait_store(bn)
            start_gather(j0 + b + LA, bn)
        wait_gather(b)
        start_store(j0 + b, b)

    for b in range(NBUF):
        wait_store(b)


def kernel(positions, encodings):
    idx = positions.reshape(NW, NGROUP, GROUP).astype(jnp.int32)
    out = _gather_kernel(encodings, idx)
    return out.reshape(positions.shape[0], 1, positions.shape[1], DIM)


# speedup vs baseline: 1.0651x; 1.0008x over previous
import functools

import jax
import jax.numpy as jnp
from jax import lax
from jax.experimental import pallas as pl
from jax.experimental.pallas import tpu as pltpu
from jax.experimental.pallas import tpu_sc as plsc

DIM = 128
ROWS = 8192
N_IDX = 4096 * 50
GROUP = 80
_info = plsc.get_sparse_core_info()
NC = _info.num_cores
NS = _info.num_subcores
NW = NC * NS
PER_W = N_IDX // NW
NGROUP = PER_W // GROUP
NBUF = 5
LA = 2
NB = NGROUP // NBUF
STAGE = ROWS // NS


@functools.partial(
    pl.kernel,
    out_type=jax.ShapeDtypeStruct((N_IDX, DIM), jnp.float32),
    mesh=plsc.VectorSubcoreMesh(core_axis_name="c", subcore_axis_name="s"),
    scratch_types=[
        pltpu.VMEM_SHARED((ROWS, DIM), jnp.float32),
        pltpu.VMEM((NGROUP, GROUP), jnp.int32),
        [pltpu.VMEM((GROUP, DIM), jnp.float32)] * NBUF,
        [pltpu.SemaphoreType.DMA] * NBUF,
        [pltpu.SemaphoreType.DMA] * NBUF,
        pltpu.SemaphoreType.DMA,
    ],
)
def _gather_kernel(table_hbm, idx_hbm, out_hbm, table_sp, idx_v, rows, gsem,
                   ssem, tsem):
    cid = lax.axis_index("c")
    sid = lax.axis_index("s")
    wid = sid * NC + cid
    row_base = wid * NGROUP

    stage_desc = pltpu.async_copy(
        table_hbm.at[pl.ds(sid * STAGE, STAGE)],
        table_sp.at[pl.ds(sid * STAGE, STAGE)],
        tsem,
    )
    pltpu.sync_copy(idx_hbm.at[wid], idx_v)
    stage_desc.wait()
    plsc.subcore_barrier()

    def start_gather(j, b):
        pltpu.async_copy(table_sp.at[idx_v.at[j]], rows[b], gsem[b])

    def wait_gather(b):
        pltpu.make_async_copy(
            table_sp.at[pl.ds(0, GROUP)], rows[b], gsem[b]
        ).wait()

    def start_store(j, b):
        pltpu.async_copy(
            rows[b], out_hbm.at[pl.ds((row_base + j) * GROUP, GROUP)], ssem[b]
        )

    def wait_store(b):
        pltpu.make_async_copy(
            rows[b], out_hbm.at[pl.ds(0, GROUP)], ssem[b]
        ).wait()

    for j in range(LA):
        start_gather(j, j)

    for b in range(NBUF):
        jn = b + LA
        if jn >= NBUF:
            wait_store(jn % NBUF)
        start_gather(jn, jn % NBUF)
        wait_gather(b)
        start_store(b, b)

    @pl.loop(1, NB - 1)
    def _(g):
        j0 = g * NBUF
        for b in range(NBUF):
            bn = (b + LA) % NBUF
            wait_store(bn)
            start_gather(j0 + b + LA, bn)
            wait_gather(b)
            start_store(j0 + b, b)

    j0 = (NB - 1) * NBUF
    for b in range(NBUF):
        if b < NBUF - LA:
            bn = (b + LA) % NBUF
            wait_store(bn)
            start_gather(j0 + b + LA, bn)
        wait_gather(b)
        start_store(j0 + b, b)

    for b in range(NBUF):
        wait_store(b)


def kernel(positions, encodings):
    idx = positions.reshape(NW, NGROUP, GROUP).astype(jnp.int32)
    out = _gather_kernel(encodings, idx)
    return out.reshape(positions.shape[0], 1, positions.shape[1], DIM)


# speedup vs baseline: 1.0677x; 1.0024x over previous
import functools

import jax
import jax.numpy as jnp
from jax import lax
from jax.experimental import pallas as pl
from jax.experimental.pallas import tpu as pltpu
from jax.experimental.pallas import tpu_sc as plsc

DIM = 128
ROWS = 8192
N_IDX = 4096 * 50
GROUP = 80
_info = plsc.get_sparse_core_info()
NC = _info.num_cores
NS = _info.num_subcores
NW = NC * NS
PER_W = N_IDX // NW
NGROUP = PER_W // GROUP
NBUF = 5
LA = 4
NB = NGROUP // NBUF
STAGE = ROWS // NS


@functools.partial(
    pl.kernel,
    out_type=jax.ShapeDtypeStruct((N_IDX, DIM), jnp.float32),
    mesh=plsc.VectorSubcoreMesh(core_axis_name="c", subcore_axis_name="s"),
    scratch_types=[
        pltpu.VMEM_SHARED((ROWS, DIM), jnp.float32),
        pltpu.VMEM((NGROUP, GROUP), jnp.int32),
        [pltpu.VMEM((GROUP, DIM), jnp.float32)] * NBUF,
        [pltpu.SemaphoreType.DMA] * NBUF,
        [pltpu.SemaphoreType.DMA] * NBUF,
        pltpu.SemaphoreType.DMA,
    ],
)
def _gather_kernel(table_hbm, idx_hbm, out_hbm, table_sp, idx_v, rows, gsem,
                   ssem, tsem):
    cid = lax.axis_index("c")
    sid = lax.axis_index("s")
    wid = sid * NC + cid
    row_base = wid * NGROUP

    stage_desc = pltpu.async_copy(
        table_hbm.at[pl.ds(sid * STAGE, STAGE)],
        table_sp.at[pl.ds(sid * STAGE, STAGE)],
        tsem,
    )
    pltpu.sync_copy(idx_hbm.at[wid], idx_v)
    stage_desc.wait()
    plsc.subcore_barrier()

    def start_gather(j, b):
        pltpu.async_copy(table_sp.at[idx_v.at[j]], rows[b], gsem[b])

    def wait_gather(b):
        pltpu.make_async_copy(
            table_sp.at[pl.ds(0, GROUP)], rows[b], gsem[b]
        ).wait()

    def start_store(j, b):
        pltpu.async_copy(
            rows[b], out_hbm.at[pl.ds((row_base + j) * GROUP, GROUP)], ssem[b]
        )

    def wait_store(b):
        pltpu.make_async_copy(
            rows[b], out_hbm.at[pl.ds(0, GROUP)], ssem[b]
        ).wait()

    for j in range(LA):
        start_gather(j, j)

    for b in range(NBUF):
        jn = b + LA
        if jn >= NBUF:
            wait_store(jn % NBUF)
        start_gather(jn, jn % NBUF)
        wait_gather(b)
        start_store(b, b)

    @pl.loop(1, NB - 1)
    def _(g):
        j0 = g * NBUF
        for b in range(NBUF):
            bn = (b + LA) % NBUF
            wait_store(bn)
            start_gather(j0 + b + LA, bn)
            wait_gather(b)
            start_store(j0 + b, b)

    j0 = (NB - 1) * NBUF
    for b in range(NBUF):
        if b < NBUF - LA:
            bn = (b + LA) % NBUF
            wait_store(bn)
            start_gather(j0 + b + LA, bn)
        wait_gather(b)
        start_store(j0 + b, b)

    for b in range(NBUF):
        wait_store(b)


def kernel(positions, encodings):
    idx = positions.reshape(NW, NGROUP, GROUP).astype(jnp.int32)
    out = _gather_kernel(encodings, idx)
    return out.reshape(positions.shape[0], 1, positions.shape[1], DIM)
